# baseline (speedup 1.0000x reference)
import jax
import jax.numpy as jnp
from jax import lax
from jax.experimental import pallas as pl
from jax.experimental.pallas import tpu as pltpu
from jax.experimental.pallas import tpu_sc as plsc

_N = 10000
_E = 320000
_R = 8
_D = 128
_RN = _R * _N
_NC = 2
_NS = 16
_NW = _NC * _NS
_EW = _E // _NW
_C = 80
_NCH = _EW // _C
_L = 16
_BN = 1000
_RB = 624
_RREM = _N - _NS * _RB
_PART = 5
_EPP = _EW // _PART
_CPP = _EPP // _C


def _deg_body(dstf, etf, degp, hist, dstb, etb):
    c = lax.axis_index("c")
    s = lax.axis_index("s")
    wid = s * _NC + c
    pltpu.sync_copy(dstf.at[pl.ds(wid * _EW, _EW)], dstb)
    pltpu.sync_copy(etf.at[pl.ds(wid * _EW, _EW)], etb)
    zeros = jnp.zeros((_L,), jnp.int32)

    def _zero(i, carry):
        for u in range(10):
            hist[pl.ds((i * 10 + u) * _L, _L)] = zeros
        return carry

    lax.fori_loop(0, _RN // _L // 10, _zero, 0)
    ones = jnp.ones((_L,), jnp.int32)

    def _edges(i, carry):
        for u in range(5):
            sl = pl.ds((i * 5 + u) * _L, _L)
            seg = etb[sl] * _N + dstb[sl]
            plsc.addupdate_scatter(hist, [seg], ones)
        return carry

    lax.fori_loop(0, _EW // _L // 5, _edges, 0)
    pltpu.sync_copy(hist, degp.at[pl.ds(wid * _RN, _RN)])


def _agg_body(tt, winv, srcf, dstf, etf, outp,
              srcb, dstb, etb, rows0, rows1, fidxb0, fidxb1,
              didxb0, didxb1, widxb0, widxb1, wbuf0, wbuf1,
              out_sh, winv_sh, semr0, semr1, semw0, semw1, sems0, sems1):
    c = lax.axis_index("c")
    s = lax.axis_index("s")
    wid = s * _NC + c
    rows = (rows0, rows1)
    fidxb = (fidxb0, fidxb1)
    didxb = (didxb0, didxb1)
    widxb = (widxb0, widxb1)
    wbuf = (wbuf0, wbuf1)
    semr = (semr0, semr1)
    semw = (semw0, semw1)
    sems = (sems0, sems1)

    @pl.when(s == 0)
    def _stage_winv():
        pltpu.sync_copy(winv, winv_sh)

    zf = jnp.zeros((_L,), jnp.float32)
    zi = jnp.zeros((_L,), jnp.int32)

    def _zr(k, cc):
        for i16 in range(_L):
            i = k * _L + i16
            for q in range(_D // _L):
                qs = pl.ds(q * _L, _L)
                rows0[i, qs] = zf
                rows1[i, qs] = zf
        return cc

    lax.fori_loop(0, _C // _L, _zr, 0)
    for k in range(_C // _L):
        didxb0[pl.ds(k * _L, _L)] = zi
        didxb1[pl.ds(k * _L, _L)] = zi
    pltpu.async_copy(rows0, out_sh.at[didxb0], sems0, add=True)
    pltpu.async_copy(rows1, out_sh.at[didxb1], sems1, add=True)

    def _zo(m, carry):
        pltpu.sync_copy(rows0.at[pl.ds(0, _RREM)],
                        out_sh.at[pl.ds(s * _RB + m * _RREM, _RREM)])
        return carry

    lax.fori_loop(0, _RB // _RREM, _zo, 0)

    @pl.when(s == _NS - 1)
    def _zlast():
        pltpu.sync_copy(rows0.at[pl.ds(0, _RREM)],
                        out_sh.at[pl.ds(_NS * _RB, _RREM)])

    plsc.subcore_barrier()

    def _fire(cc, p):
        pltpu.make_async_copy(rows[p], out_sh.at[didxb[p]], sems[p]).wait()
        base = cc * _C
        for k in range(_C // _L):
            sl = pl.ds(k * _L, _L)
            esl = pl.ds(base + k * _L, _L)
            tv = etb[esl]
            dv = dstb[esl]
            fidxb[p][sl] = tv * _N + srcb[esl]
            widxb[p][sl] = tv * _N + dv
            didxb[p][sl] = dv
        pltpu.async_copy(tt.at[fidxb[p]], rows[p], semr[p])
        pltpu.async_copy(winv_sh.at[widxb[p]], wbuf[p], semw[p])

    def _process(p):
        pltpu.make_async_copy(tt.at[fidxb[p]], rows[p], semr[p]).wait()
        pltpu.make_async_copy(winv_sh.at[widxb[p]], wbuf[p],
                              semw[p]).wait()
        def _scale(k, cc):
            wvec = wbuf[p][pl.ds(k * _L, _L)]
            for i16 in range(_L):
                i = k * _L + i16
                wv = jnp.full((_L,), wvec[i16], jnp.float32)
                for q in range(_D // _L):
                    qs = pl.ds(q * _L, _L)
                    rows[p][i, qs] = rows[p][i, qs] * wv
            return cc

        lax.fori_loop(0, _C // _L, _scale, 0)
        pltpu.async_copy(rows[p], out_sh.at[didxb[p]], sems[p], add=True)

    for h in range(_PART):
        off = wid * _EW + h * _EPP
        pltpu.sync_copy(srcf.at[pl.ds(off, _EPP)], srcb)
        pltpu.sync_copy(dstf.at[pl.ds(off, _EPP)], dstb)
        pltpu.sync_copy(etf.at[pl.ds(off, _EPP)], etb)
        _fire(0, 0)

        def _pair(j, carry):
            _fire(2 * j + 1, 1)
            _process(0)
            _fire(2 * j + 2, 0)
            _process(1)
            return carry

        lax.fori_loop(0, (_CPP - 1) // 2, _pair, 0)
        _process(0)

    pltpu.make_async_copy(rows[0], out_sh.at[didxb[0]], sems[0]).wait()
    pltpu.make_async_copy(rows[1], out_sh.at[didxb[1]], sems[1]).wait()
    plsc.subcore_barrier()
    pltpu.sync_copy(out_sh.at[pl.ds(s * _RB, _RB)],
                    outp.at[c, pl.ds(s * _RB, _RB)])

    @pl.when(s == _NS - 1)
    def _clast():
        pltpu.sync_copy(out_sh.at[pl.ds(_NS * _RB, _RREM)],
                        outp.at[c, pl.ds(_NS * _RB, _RREM)])


def _mmw_body(x_ref, w_ref, dp_ref, tt_ref, wv_ref):
    tt_ref[...] = jnp.dot(x_ref[...], w_ref[0],
                          preferred_element_type=jnp.float32)

    @pl.when(jnp.logical_and(pl.program_id(0) == 0, pl.program_id(1) == 0))
    def _winv():
        ssum = jnp.sum(dp_ref[...], axis=0).astype(jnp.float32)
        wv_ref[...] = (1.0 / jnp.maximum(ssum, 1.0))[None, :]


def _fin_body(p_ref, b_ref, o_ref):
    o_ref[...] = p_ref[0] + p_ref[1] + b_ref[...]


def kernel(x, edge_index, edge_type, W, b):
    srcf = edge_index[0]
    dstf = edge_index[1]
    etf = edge_type

    mesh = plsc.VectorSubcoreMesh(core_axis_name="c", subcore_axis_name="s")

    degp = pl.kernel(
        _deg_body,
        out_type=jax.ShapeDtypeStruct((_NW * _RN,), jnp.int32),
        mesh=mesh,
        scratch_types=[
            pltpu.VMEM((_RN,), jnp.int32),
            pltpu.VMEM((_EW,), jnp.int32),
            pltpu.VMEM((_EW,), jnp.int32),
        ],
        compiler_params=pltpu.CompilerParams(needs_layout_passes=False),
    )(dstf, etf)

    tt, winv2 = pl.pallas_call(
        _mmw_body,
        grid=(_R, _N // _BN),
        in_specs=[
            pl.BlockSpec((_BN, _D), lambda r, n: (n, 0)),
            pl.BlockSpec((1, _D, _D), lambda r, n: (r, 0, 0)),
            pl.BlockSpec((_NW, _RN), lambda r, n: (0, 0)),
        ],
        out_specs=[
            pl.BlockSpec((_BN, _D), lambda r, n: (r * (_N // _BN) + n, 0)),
            pl.BlockSpec((1, _RN), lambda r, n: (0, 0)),
        ],
        out_shape=[
            jax.ShapeDtypeStruct((_RN, _D), jnp.float32),
            jax.ShapeDtypeStruct((1, _RN), jnp.float32),
        ],
    )(x, W, degp.reshape(_NW, _RN))
    winv = winv2.reshape(_RN)

    outp = pl.kernel(
        _agg_body,
        out_type=jax.ShapeDtypeStruct((_NC, _N, _D), jnp.float32),
        mesh=mesh,
        scratch_types=[
            pltpu.VMEM((_EPP,), jnp.int32),
            pltpu.VMEM((_EPP,), jnp.int32),
            pltpu.VMEM((_EPP,), jnp.int32),
            pltpu.VMEM((_C, _D), jnp.float32),
            pltpu.VMEM((_C, _D), jnp.float32),
            pltpu.VMEM((_C,), jnp.int32),
            pltpu.VMEM((_C,), jnp.int32),
            pltpu.VMEM((_C,), jnp.int32),
            pltpu.VMEM((_C,), jnp.int32),
            pltpu.VMEM((_C,), jnp.int32),
            pltpu.VMEM((_C,), jnp.int32),
            pltpu.VMEM((_C,), jnp.float32),
            pltpu.VMEM((_C,), jnp.float32),
            pltpu.VMEM_SHARED((_N, _D), jnp.float32),
            pltpu.VMEM_SHARED((_RN,), jnp.float32),
            pltpu.SemaphoreType.DMA,
            pltpu.SemaphoreType.DMA,
            pltpu.SemaphoreType.DMA,
            pltpu.SemaphoreType.DMA,
            pltpu.SemaphoreType.DMA,
            pltpu.SemaphoreType.DMA,
        ],
        compiler_params=pltpu.CompilerParams(needs_layout_passes=False),
    )(tt, winv, srcf, dstf, etf)

    out = pl.pallas_call(
        _fin_body,
        grid=(_N // _BN,),
        in_specs=[
            pl.BlockSpec((_NC, _BN, _D), lambda i: (0, i, 0)),
            pl.BlockSpec((1, _D), lambda i: (0, 0)),
        ],
        out_specs=pl.BlockSpec((_BN, _D), lambda i: (i, 0)),
        out_shape=jax.ShapeDtypeStruct((_N, _D), jnp.float32),
    )(outp, b.reshape(1, _D))
    return out


# speedup vs baseline: 1.0004x; 1.0004x over previous
import jax
import jax.numpy as jnp
from jax import lax
from jax.experimental import pallas as pl
from jax.experimental.pallas import tpu as pltpu
from jax.experimental.pallas import tpu_sc as plsc

_N = 10000
_E = 320000
_R = 8
_D = 128
_RN = _R * _N
_NC = 2
_NS = 16
_NW = _NC * _NS
_EW = _E // _NW
_C = 80
_NCH = _EW // _C
_L = 16
_BN = 1000
_RB = 624
_RREM = _N - _NS * _RB
_PART = 5
_EPP = _EW // _PART
_CPP = _EPP // _C
_HR = 640
_HC = 128
_RNP = _HR * _HC


def _deg_body(dstf, etf, degp, hist2, dstb, etb, idxb, hist_sh):
    c = lax.axis_index("c")
    s = lax.axis_index("s")
    wid = s * _NC + c
    pltpu.sync_copy(dstf.at[pl.ds(wid * _EW, _EW)], dstb)
    pltpu.sync_copy(etf.at[pl.ds(wid * _EW, _EW)], etb)
    zeros = jnp.zeros((_L,), jnp.int32)

    def _zero(i, carry):
        for u in range(8):
            for q in range(_HC // _L):
                hist2[i * 8 + u, pl.ds(q * _L, _L)] = zeros
        return carry

    lax.fori_loop(0, _HR // 8, _zero, 0)

    @pl.when(s == 0)
    def _zero_shared():
        pltpu.sync_copy(hist2, hist_sh)

    plsc.subcore_barrier()
    ones = jnp.ones((_L,), jnp.int32)

    def _edges(i, carry):
        for u in range(5):
            sl = pl.ds((i * 5 + u) * _L, _L)
            seg = etb[sl] * _N + dstb[sl]
            plsc.addupdate_scatter(hist2, [seg >> 7, seg & 127], ones)
        return carry

    lax.fori_loop(0, _EW // _L // 5, _edges, 0)
    for m in range(_HR // _HC):
        for u in range(_HC // _L):
            idxb[pl.ds(u * _L, _L)] = (lax.iota(jnp.int32, _L)
                                       + (m * _HC + u * _L))
        pltpu.sync_copy(hist2.at[pl.ds(m * _HC, _HC)],
                        hist_sh.at[idxb], add=True)
    plsc.subcore_barrier()
    rpt = _HR // _NS
    pltpu.sync_copy(hist_sh.at[pl.ds(s * rpt, rpt)],
                    degp.at[c, pl.ds(s * rpt, rpt)])


def _agg_body(tt, winv, srcf, dstf, etf, outp,
              srcb, dstb, etb, rows0, rows1, fidxb0, fidxb1,
              didxb0, didxb1, widxb0, widxb1, wbuf0, wbuf1,
              out_sh, winv_sh, semr0, semr1, semw0, semw1, sems0, sems1):
    c = lax.axis_index("c")
    s = lax.axis_index("s")
    wid = s * _NC + c
    rows = (rows0, rows1)
    fidxb = (fidxb0, fidxb1)
    didxb = (didxb0, didxb1)
    widxb = (widxb0, widxb1)
    wbuf = (wbuf0, wbuf1)
    semr = (semr0, semr1)
    semw = (semw0, semw1)
    sems = (sems0, sems1)

    @pl.when(s == 0)
    def _stage_winv():
        pltpu.sync_copy(winv, winv_sh)

    zf = jnp.zeros((_L,), jnp.float32)
    zi = jnp.zeros((_L,), jnp.int32)

    def _zr(k, cc):
        for i16 in range(_L):
            i = k * _L + i16
            for q in range(_D // _L):
                qs = pl.ds(q * _L, _L)
                rows0[i, qs] = zf
                rows1[i, qs] = zf
        return cc

    lax.fori_loop(0, _C // _L, _zr, 0)
    for k in range(_C // _L):
        didxb0[pl.ds(k * _L, _L)] = zi
        didxb1[pl.ds(k * _L, _L)] = zi
    pltpu.async_copy(rows0, out_sh.at[didxb0], sems0, add=True)
    pltpu.async_copy(rows1, out_sh.at[didxb1], sems1, add=True)

    def _zo(m, carry):
        pltpu.sync_copy(rows0.at[pl.ds(0, _RREM)],
                        out_sh.at[pl.ds(s * _RB + m * _RREM, _RREM)])
        return carry

    lax.fori_loop(0, _RB // _RREM, _zo, 0)

    @pl.when(s == _NS - 1)
    def _zlast():
        pltpu.sync_copy(rows0.at[pl.ds(0, _RREM)],
                        out_sh.at[pl.ds(_NS * _RB, _RREM)])

    plsc.subcore_barrier()

    def _fire(cc, p):
        pltpu.make_async_copy(rows[p], out_sh.at[didxb[p]], sems[p]).wait()
        base = cc * _C
        for k in range(_C // _L):
            sl = pl.ds(k * _L, _L)
            esl = pl.ds(base + k * _L, _L)
            tv = etb[esl]
            dv = dstb[esl]
            fidxb[p][sl] = tv * _N + srcb[esl]
            widxb[p][sl] = tv * _N + dv
            didxb[p][sl] = dv
        pltpu.async_copy(tt.at[fidxb[p]], rows[p], semr[p])
        pltpu.async_copy(winv_sh.at[widxb[p]], wbuf[p], semw[p])

    def _process(p):
        pltpu.make_async_copy(tt.at[fidxb[p]], rows[p], semr[p]).wait()
        pltpu.make_async_copy(winv_sh.at[widxb[p]], wbuf[p],
                              semw[p]).wait()
        def _scale(k, cc):
            wvec = wbuf[p][pl.ds(k * _L, _L)]
            for i16 in range(_L):
                i = k * _L + i16
                wv = jnp.full((_L,), wvec[i16], jnp.float32)
                for q in range(_D // _L):
                    qs = pl.ds(q * _L, _L)
                    rows[p][i, qs] = rows[p][i, qs] * wv
            return cc

        lax.fori_loop(0, _C // _L, _scale, 0)
        pltpu.async_copy(rows[p], out_sh.at[didxb[p]], sems[p], add=True)

    for h in range(_PART):
        off = wid * _EW + h * _EPP
        pltpu.sync_copy(srcf.at[pl.ds(off, _EPP)], srcb)
        pltpu.sync_copy(dstf.at[pl.ds(off, _EPP)], dstb)
        pltpu.sync_copy(etf.at[pl.ds(off, _EPP)], etb)
        _fire(0, 0)

        def _pair(j, carry):
            _fire(2 * j + 1, 1)
            _process(0)
            _fire(2 * j + 2, 0)
            _process(1)
            return carry

        lax.fori_loop(0, (_CPP - 1) // 2, _pair, 0)
        _process(0)

    pltpu.make_async_copy(rows[0], out_sh.at[didxb[0]], sems[0]).wait()
    pltpu.make_async_copy(rows[1], out_sh.at[didxb[1]], sems[1]).wait()
    plsc.subcore_barrier()
    pltpu.sync_copy(out_sh.at[pl.ds(s * _RB, _RB)],
                    outp.at[c, pl.ds(s * _RB, _RB)])

    @pl.when(s == _NS - 1)
    def _clast():
        pltpu.sync_copy(out_sh.at[pl.ds(_NS * _RB, _RREM)],
                        outp.at[c, pl.ds(_NS * _RB, _RREM)])


def _mmw_body(x_ref, w_ref, dp_ref, tt_ref, wv_ref):
    tt_ref[...] = jnp.dot(x_ref[...], w_ref[0],
                          preferred_element_type=jnp.float32)

    @pl.when(jnp.logical_and(pl.program_id(0) == 0, pl.program_id(1) == 0))
    def _winv():
        ssum = jnp.sum(dp_ref[...], axis=0).astype(jnp.float32)
        wv_ref[...] = (1.0 / jnp.maximum(ssum, 1.0))[None, :]


def _fin_body(p_ref, b_ref, o_ref):
    o_ref[...] = p_ref[0] + p_ref[1] + b_ref[...]


def kernel(x, edge_index, edge_type, W, b):
    srcf = edge_index[0]
    dstf = edge_index[1]
    etf = edge_type

    mesh = plsc.VectorSubcoreMesh(core_axis_name="c", subcore_axis_name="s")

    degp = pl.kernel(
        _deg_body,
        out_type=jax.ShapeDtypeStruct((_NC, _HR, _HC), jnp.int32),
        mesh=mesh,
        scratch_types=[
            pltpu.VMEM((_HR, _HC), jnp.int32),
            pltpu.VMEM((_EW,), jnp.int32),
            pltpu.VMEM((_EW,), jnp.int32),
            pltpu.VMEM((_HC,), jnp.int32),
            pltpu.VMEM_SHARED((_HR, _HC), jnp.int32),
        ],
        compiler_params=pltpu.CompilerParams(needs_layout_passes=False),
    )(dstf, etf)

    tt, winv2 = pl.pallas_call(
        _mmw_body,
        grid=(_R, _N // _BN),
        in_specs=[
            pl.BlockSpec((_BN, _D), lambda r, n: (n, 0)),
            pl.BlockSpec((1, _D, _D), lambda r, n: (r, 0, 0)),
            pl.BlockSpec((_NC, _RNP), lambda r, n: (0, 0)),
        ],
        out_specs=[
            pl.BlockSpec((_BN, _D), lambda r, n: (r * (_N // _BN) + n, 0)),
            pl.BlockSpec((1, _RNP), lambda r, n: (0, 0)),
        ],
        out_shape=[
            jax.ShapeDtypeStruct((_RN, _D), jnp.float32),
            jax.ShapeDtypeStruct((1, _RNP), jnp.float32),
        ],
    )(x, W, degp.reshape(_NC, _RNP))
    winv = winv2.reshape(_RNP)

    outp = pl.kernel(
        _agg_body,
        out_type=jax.ShapeDtypeStruct((_NC, _N, _D), jnp.float32),
        mesh=mesh,
        scratch_types=[
            pltpu.VMEM((_EPP,), jnp.int32),
            pltpu.VMEM((_EPP,), jnp.int32),
            pltpu.VMEM((_EPP,), jnp.int32),
            pltpu.VMEM((_C, _D), jnp.float32),
            pltpu.VMEM((_C, _D), jnp.float32),
            pltpu.VMEM((_C,), jnp.int32),
            pltpu.VMEM((_C,), jnp.int32),
            pltpu.VMEM((_C,), jnp.int32),
            pltpu.VMEM((_C,), jnp.int32),
            pltpu.VMEM((_C,), jnp.int32),
            pltpu.VMEM((_C,), jnp.int32),
            pltpu.VMEM((_C,), jnp.float32),
            pltpu.VMEM((_C,), jnp.float32),
            pltpu.VMEM_SHARED((_N, _D), jnp.float32),
            pltpu.VMEM_SHARED((_RNP,), jnp.float32),
            pltpu.SemaphoreType.DMA,
            pltpu.SemaphoreType.DMA,
            pltpu.SemaphoreType.DMA,
            pltpu.SemaphoreType.DMA,
            pltpu.SemaphoreType.DMA,
            pltpu.SemaphoreType.DMA,
        ],
        compiler_params=pltpu.CompilerParams(needs_layout_passes=False),
    )(tt, winv, srcf, dstf, etf)

    out = pl.pallas_call(
        _fin_body,
        grid=(_N // _BN,),
        in_specs=[
            pl.BlockSpec((_NC, _BN, _D), lambda i: (0, i, 0)),
            pl.BlockSpec((1, _D), lambda i: (0, 0)),
        ],
        out_specs=pl.BlockSpec((_BN, _D), lambda i: (i, 0)),
        out_shape=jax.ShapeDtypeStruct((_N, _D), jnp.float32),
    )(outp, b.reshape(1, _D))
    return out


# speedup vs baseline: 1.0668x; 1.0664x over previous
import jax
import jax.numpy as jnp
from jax import lax
from jax.experimental import pallas as pl
from jax.experimental.pallas import tpu as pltpu
from jax.experimental.pallas import tpu_sc as plsc

_N = 10000
_E = 320000
_R = 8
_D = 128
_RN = _R * _N
_NC = 2
_NS = 16
_NW = _NC * _NS
_EW = _E // _NW
_C = 80
_NCH = _EW // _C
_L = 16
_BN = 1000
_RB = 624
_RREM = _N - _NS * _RB
_PART = 5
_EPP = _EW // _PART
_CPP = _EPP // _C
_HR = 640
_HC = 128
_RNP = _HR * _HC


def _deg_body(dstf, etf, degp, hist2, dstb, etb, idxb, hist_sh):
    c = lax.axis_index("c")
    s = lax.axis_index("s")
    wid = s * _NC + c
    pltpu.sync_copy(dstf.at[pl.ds(wid * _EW, _EW)], dstb)
    pltpu.sync_copy(etf.at[pl.ds(wid * _EW, _EW)], etb)
    zeros = jnp.zeros((_L,), jnp.int32)

    def _zero(i, carry):
        for u in range(8):
            for q in range(_HC // _L):
                hist2[i * 8 + u, pl.ds(q * _L, _L)] = zeros
        return carry

    lax.fori_loop(0, _HR // 8, _zero, 0)

    @pl.when(s == 0)
    def _zero_shared():
        pltpu.sync_copy(hist2, hist_sh)

    plsc.subcore_barrier()
    ones = jnp.ones((_L,), jnp.int32)

    def _edges(i, carry):
        for u in range(5):
            sl = pl.ds((i * 5 + u) * _L, _L)
            seg = etb[sl] * _N + dstb[sl]
            plsc.addupdate_scatter(hist2, [seg >> 7, seg & 127], ones)
        return carry

    lax.fori_loop(0, _EW // _L // 5, _edges, 0)
    for m in range(_HR // _HC):
        for u in range(_HC // _L):
            idxb[pl.ds(u * _L, _L)] = (lax.iota(jnp.int32, _L)
                                       + (m * _HC + u * _L))
        pltpu.sync_copy(hist2.at[pl.ds(m * _HC, _HC)],
                        hist_sh.at[idxb], add=True)
    plsc.subcore_barrier()
    rpt = _HR // _NS
    pltpu.sync_copy(hist_sh.at[pl.ds(s * rpt, rpt)],
                    degp.at[c, pl.ds(s * rpt, rpt)])


def _agg_body(tt, winv, srcf, dstf, etf, outp,
              srcb, dstb, etb, rows0, rows1, fidxb0, fidxb1,
              didxb0, didxb1, widxb0, widxb1, wbuf0, wbuf1,
              out_sh, winv_sh, semr0, semr1, semw0, semw1):
    c = lax.axis_index("c")
    s = lax.axis_index("s")
    wid = s * _NC + c
    rows = (rows0, rows1)
    fidxb = (fidxb0, fidxb1)
    didxb = (didxb0, didxb1)
    widxb = (widxb0, widxb1)
    wbuf = (wbuf0, wbuf1)
    semr = (semr0, semr1)
    semw = (semw0, semw1)

    @pl.when(s == 0)
    def _stage_winv():
        pltpu.sync_copy(winv, winv_sh)

    zf = jnp.zeros((_L,), jnp.float32)
    for i in range(_RREM):
        for q in range(_D // _L):
            rows0[i, pl.ds(q * _L, _L)] = zf

    def _zo(m, carry):
        pltpu.sync_copy(rows0.at[pl.ds(0, _RREM)],
                        out_sh.at[pl.ds(s * _RB + m * _RREM, _RREM)])
        return carry

    lax.fori_loop(0, _RB // _RREM, _zo, 0)

    @pl.when(s == _NS - 1)
    def _zlast():
        pltpu.sync_copy(rows0.at[pl.ds(0, _RREM)],
                        out_sh.at[pl.ds(_NS * _RB, _RREM)])

    plsc.subcore_barrier()

    def _fire(cc, p):
        base = cc * _C
        for k in range(_C // _L):
            sl = pl.ds(k * _L, _L)
            esl = pl.ds(base + k * _L, _L)
            tv = etb[esl]
            dv = dstb[esl]
            fidxb[p][sl] = tv * _N + srcb[esl]
            widxb[p][sl] = tv * _N + dv
            didxb[p][sl] = dv
        pltpu.async_copy(tt.at[fidxb[p]], rows[p], semr[p])
        pltpu.async_copy(winv_sh.at[widxb[p]], wbuf[p], semw[p])

    def _process(p):
        pltpu.make_async_copy(tt.at[fidxb[p]], rows[p], semr[p]).wait()
        pltpu.make_async_copy(winv_sh.at[widxb[p]], wbuf[p],
                              semw[p]).wait()
        def _scale(k, cc):
            wvec = wbuf[p][pl.ds(k * _L, _L)]
            for i16 in range(_L):
                i = k * _L + i16
                wv = jnp.full((_L,), wvec[i16], jnp.float32)
                for q in range(_D // _L):
                    qs = pl.ds(q * _L, _L)
                    rows[p][i, qs] = rows[p][i, qs] * wv
            return cc

        lax.fori_loop(0, _C // _L, _scale, 0)
        pltpu.sync_copy(rows[p], out_sh.at[didxb[p]], add=True)

    for h in range(_PART):
        off = wid * _EW + h * _EPP
        pltpu.sync_copy(srcf.at[pl.ds(off, _EPP)], srcb)
        pltpu.sync_copy(dstf.at[pl.ds(off, _EPP)], dstb)
        pltpu.sync_copy(etf.at[pl.ds(off, _EPP)], etb)
        _fire(0, 0)

        def _pair(j, carry):
            _fire(2 * j + 1, 1)
            _process(0)
            _fire(2 * j + 2, 0)
            _process(1)
            return carry

        lax.fori_loop(0, (_CPP - 1) // 2, _pair, 0)
        _process(0)

    plsc.subcore_barrier()
    pltpu.sync_copy(out_sh.at[pl.ds(s * _RB, _RB)],
                    outp.at[c, pl.ds(s * _RB, _RB)])

    @pl.when(s == _NS - 1)
    def _clast():
        pltpu.sync_copy(out_sh.at[pl.ds(_NS * _RB, _RREM)],
                        outp.at[c, pl.ds(_NS * _RB, _RREM)])


def _mm_body(x_ref, w_ref, o_ref):
    o_ref[...] = jnp.dot(x_ref[...], w_ref[0],
                         preferred_element_type=jnp.float32)


def _winv_body(dp_ref, o_ref):
    ssum = jnp.sum(dp_ref[...], axis=0).astype(jnp.float32)
    o_ref[...] = (1.0 / jnp.maximum(ssum, 1.0))[None, :]


def _fin_body(p_ref, b_ref, o_ref):
    o_ref[...] = p_ref[0] + p_ref[1] + b_ref[...]


def kernel(x, edge_index, edge_type, W, b):
    srcf = edge_index[0]
    dstf = edge_index[1]
    etf = edge_type

    mesh = plsc.VectorSubcoreMesh(core_axis_name="c", subcore_axis_name="s")

    degp = pl.kernel(
        _deg_body,
        out_type=jax.ShapeDtypeStruct((_NC, _HR, _HC), jnp.int32),
        mesh=mesh,
        scratch_types=[
            pltpu.VMEM((_HR, _HC), jnp.int32),
            pltpu.VMEM((_EW,), jnp.int32),
            pltpu.VMEM((_EW,), jnp.int32),
            pltpu.VMEM((_HC,), jnp.int32),
            pltpu.VMEM_SHARED((_HR, _HC), jnp.int32),
        ],
        compiler_params=pltpu.CompilerParams(needs_layout_passes=False),
    )(dstf, etf)

    tt = pl.pallas_call(
        _mm_body,
        grid=(_R, _N // _BN),
        in_specs=[
            pl.BlockSpec((_BN, _D), lambda r, n: (n, 0)),
            pl.BlockSpec((1, _D, _D), lambda r, n: (r, 0, 0)),
        ],
        out_specs=pl.BlockSpec((_BN, _D),
                               lambda r, n: (r * (_N // _BN) + n, 0)),
        out_shape=jax.ShapeDtypeStruct((_RN, _D), jnp.float32),
    )(x, W)

    winv2 = pl.pallas_call(
        _winv_body,
        out_shape=jax.ShapeDtypeStruct((1, _RNP), jnp.float32),
    )(degp.reshape(_NC, _RNP))
    winv = winv2.reshape(_RNP)

    outp = pl.kernel(
        _agg_body,
        out_type=jax.ShapeDtypeStruct((_NC, _N, _D), jnp.float32),
        mesh=mesh,
        scratch_types=[
            pltpu.VMEM((_EPP,), jnp.int32),
            pltpu.VMEM((_EPP,), jnp.int32),
            pltpu.VMEM((_EPP,), jnp.int32),
            pltpu.VMEM((_C, _D), jnp.float32),
            pltpu.VMEM((_C, _D), jnp.float32),
            pltpu.VMEM((_C,), jnp.int32),
            pltpu.VMEM((_C,), jnp.int32),
            pltpu.VMEM((_C,), jnp.int32),
            pltpu.VMEM((_C,), jnp.int32),
            pltpu.VMEM((_C,), jnp.int32),
            pltpu.VMEM((_C,), jnp.int32),
            pltpu.VMEM((_C,), jnp.float32),
            pltpu.VMEM((_C,), jnp.float32),
            pltpu.VMEM_SHARED((_N, _D), jnp.float32),
            pltpu.VMEM_SHARED((_RNP,), jnp.float32),
            pltpu.SemaphoreType.DMA,
            pltpu.SemaphoreType.DMA,
            pltpu.SemaphoreType.DMA,
            pltpu.SemaphoreType.DMA,
        ],
        compiler_params=pltpu.CompilerParams(needs_layout_passes=False),
    )(tt, winv, srcf, dstf, etf)

    out = pl.pallas_call(
        _fin_body,
        grid=(_N // _BN,),
        in_specs=[
            pl.BlockSpec((_NC, _BN, _D), lambda i: (0, i, 0)),
            pl.BlockSpec((1, _D), lambda i: (0, 0)),
        ],
        out_specs=pl.BlockSpec((_BN, _D), lambda i: (i, 0)),
        out_shape=jax.ShapeDtypeStruct((_N, _D), jnp.float32),
    )(outp, b.reshape(1, _D))
    return out


# speedup vs baseline: 1.0685x; 1.0016x over previous
import jax
import jax.numpy as jnp
from jax import lax
from jax.experimental import pallas as pl
from jax.experimental.pallas import tpu as pltpu
from jax.experimental.pallas import tpu_sc as plsc

_N = 10000
_E = 320000
_R = 8
_D = 128
_RN = _R * _N
_NC = 2
_NS = 16
_NW = _NC * _NS
_EW = _E // _NW
_C = 80
_NCH = _EW // _C
_L = 16
_BN = 1000
_RB = 624
_RREM = _N - _NS * _RB
_PART = 5
_EPP = _EW // _PART
_CPP = _EPP // _C
_HR = 640
_HC = 128
_RNP = _HR * _HC


def _deg_body(dstf, etf, degp, hist2, dstb, etb, idxb, hist_sh):
    c = lax.axis_index("c")
    s = lax.axis_index("s")
    wid = s * _NC + c
    pltpu.sync_copy(dstf.at[pl.ds(wid * _EW, _EW)], dstb)
    pltpu.sync_copy(etf.at[pl.ds(wid * _EW, _EW)], etb)
    zeros = jnp.zeros((_L,), jnp.int32)

    def _zero(i, carry):
        for u in range(8):
            for q in range(_HC // _L):
                hist2[i * 8 + u, pl.ds(q * _L, _L)] = zeros
        return carry

    lax.fori_loop(0, _HR // 8, _zero, 0)

    @pl.when(s == 0)
    def _zero_shared():
        pltpu.sync_copy(hist2, hist_sh)

    plsc.subcore_barrier()
    ones = jnp.ones((_L,), jnp.int32)

    def _edges(i, carry):
        for u in range(5):
            sl = pl.ds((i * 5 + u) * _L, _L)
            seg = etb[sl] * _N + dstb[sl]
            plsc.addupdate_scatter(hist2, [seg >> 7, seg & 127], ones)
        return carry

    lax.fori_loop(0, _EW // _L // 5, _edges, 0)
    for m in range(_HR // _HC):
        for u in range(_HC // _L):
            idxb[pl.ds(u * _L, _L)] = (lax.iota(jnp.int32, _L)
                                       + (m * _HC + u * _L))
        pltpu.sync_copy(hist2.at[pl.ds(m * _HC, _HC)],
                        hist_sh.at[idxb], add=True)
    plsc.subcore_barrier()
    rpt = _HR // _NS
    pltpu.sync_copy(hist_sh.at[pl.ds(s * rpt, rpt)],
                    degp.at[c, pl.ds(s * rpt, rpt)])


def _agg_body(tt, degp0, degp1, srcf, dstf, etf, outp,
              srcb, dstb, etb, dg0, dg1, wv_v, rows0, rows1,
              fidxb0, fidxb1, didxb0, didxb1, widxb0, widxb1,
              wbuf0, wbuf1, out_sh, winv_sh, semr0, semr1, semw0, semw1):
    c = lax.axis_index("c")
    s = lax.axis_index("s")
    wid = s * _NC + c
    rows = (rows0, rows1)
    fidxb = (fidxb0, fidxb1)
    didxb = (didxb0, didxb1)
    widxb = (widxb0, widxb1)
    wbuf = (wbuf0, wbuf1)
    semr = (semr0, semr1)
    semw = (semw0, semw1)

    wsl = _RNP // _NS
    pltpu.sync_copy(degp0.at[pl.ds(s * wsl, wsl)], dg0)
    pltpu.sync_copy(degp1.at[pl.ds(s * wsl, wsl)], dg1)

    def _wv(i, carry):
        for u in range(8):
            qs = pl.ds((i * 8 + u) * _L, _L)
            d = (dg0[qs] + dg1[qs]).astype(jnp.float32)
            wv_v[qs] = 1.0 / jnp.maximum(d, 1.0)
        return carry

    lax.fori_loop(0, wsl // _L // 8, _wv, 0)
    pltpu.sync_copy(wv_v, winv_sh.at[pl.ds(s * wsl, wsl)])

    zf = jnp.zeros((_L,), jnp.float32)
    for i in range(_RREM):
        for q in range(_D // _L):
            rows0[i, pl.ds(q * _L, _L)] = zf

    def _zo(m, carry):
        pltpu.sync_copy(rows0.at[pl.ds(0, _RREM)],
                        out_sh.at[pl.ds(s * _RB + m * _RREM, _RREM)])
        return carry

    lax.fori_loop(0, _RB // _RREM, _zo, 0)

    @pl.when(s == _NS - 1)
    def _zlast():
        pltpu.sync_copy(rows0.at[pl.ds(0, _RREM)],
                        out_sh.at[pl.ds(_NS * _RB, _RREM)])

    plsc.subcore_barrier()

    def _fire(cc, p):
        base = cc * _C
        for k in range(_C // _L):
            sl = pl.ds(k * _L, _L)
            esl = pl.ds(base + k * _L, _L)
            tv = etb[esl]
            dv = dstb[esl]
            fidxb[p][sl] = tv * _N + srcb[esl]
            widxb[p][sl] = tv * _N + dv
            didxb[p][sl] = dv
        pltpu.async_copy(tt.at[fidxb[p]], rows[p], semr[p])
        pltpu.async_copy(winv_sh.at[widxb[p]], wbuf[p], semw[p])

    def _process(p):
        pltpu.make_async_copy(tt.at[fidxb[p]], rows[p], semr[p]).wait()
        pltpu.make_async_copy(winv_sh.at[widxb[p]], wbuf[p],
                              semw[p]).wait()
        def _scale(k, cc):
            wvec = wbuf[p][pl.ds(k * _L, _L)]
            for i16 in range(_L):
                i = k * _L + i16
                wv = jnp.full((_L,), wvec[i16], jnp.float32)
                for q in range(_D // _L):
                    qs = pl.ds(q * _L, _L)
                    rows[p][i, qs] = rows[p][i, qs] * wv
            return cc

        lax.fori_loop(0, _C // _L, _scale, 0)
        pltpu.sync_copy(rows[p], out_sh.at[didxb[p]], add=True)

    for h in range(_PART):
        off = wid * _EW + h * _EPP
        pltpu.sync_copy(srcf.at[pl.ds(off, _EPP)], srcb)
        pltpu.sync_copy(dstf.at[pl.ds(off, _EPP)], dstb)
        pltpu.sync_copy(etf.at[pl.ds(off, _EPP)], etb)
        _fire(0, 0)

        def _pair(j, carry):
            _fire(2 * j + 1, 1)
            _process(0)
            _fire(2 * j + 2, 0)
            _process(1)
            return carry

        lax.fori_loop(0, (_CPP - 1) // 2, _pair, 0)
        _process(0)

    plsc.subcore_barrier()
    pltpu.sync_copy(out_sh.at[pl.ds(s * _RB, _RB)],
                    outp.at[c, pl.ds(s * _RB, _RB)])

    @pl.when(s == _NS - 1)
    def _clast():
        pltpu.sync_copy(out_sh.at[pl.ds(_NS * _RB, _RREM)],
                        outp.at[c, pl.ds(_NS * _RB, _RREM)])


def _mm_body(x_ref, w_ref, o_ref):
    o_ref[...] = jnp.dot(x_ref[...], w_ref[0],
                         preferred_element_type=jnp.float32)


def _fin_body(p_ref, b_ref, o_ref):
    o_ref[...] = p_ref[0] + p_ref[1] + b_ref[...]


def kernel(x, edge_index, edge_type, W, b):
    srcf = edge_index[0]
    dstf = edge_index[1]
    etf = edge_type

    mesh = plsc.VectorSubcoreMesh(core_axis_name="c", subcore_axis_name="s")

    degp = pl.kernel(
        _deg_body,
        out_type=jax.ShapeDtypeStruct((_NC, _HR, _HC), jnp.int32),
        mesh=mesh,
        scratch_types=[
            pltpu.VMEM((_HR, _HC), jnp.int32),
            pltpu.VMEM((_EW,), jnp.int32),
            pltpu.VMEM((_EW,), jnp.int32),
            pltpu.VMEM((_HC,), jnp.int32),
            pltpu.VMEM_SHARED((_HR, _HC), jnp.int32),
        ],
        compiler_params=pltpu.CompilerParams(needs_layout_passes=False),
    )(dstf, etf)

    tt = pl.pallas_call(
        _mm_body,
        grid=(_R, _N // _BN),
        in_specs=[
            pl.BlockSpec((_BN, _D), lambda r, n: (n, 0)),
            pl.BlockSpec((1, _D, _D), lambda r, n: (r, 0, 0)),
        ],
        out_specs=pl.BlockSpec((_BN, _D),
                               lambda r, n: (r * (_N // _BN) + n, 0)),
        out_shape=jax.ShapeDtypeStruct((_RN, _D), jnp.float32),
    )(x, W)

    degf = degp.reshape(_NC, _RNP)

    outp = pl.kernel(
        _agg_body,
        out_type=jax.ShapeDtypeStruct((_NC, _N, _D), jnp.float32),
        mesh=mesh,
        scratch_types=[
            pltpu.VMEM((_EPP,), jnp.int32),
            pltpu.VMEM((_EPP,), jnp.int32),
            pltpu.VMEM((_EPP,), jnp.int32),
            pltpu.VMEM((_RNP // _NS,), jnp.int32),
            pltpu.VMEM((_RNP // _NS,), jnp.int32),
            pltpu.VMEM((_RNP // _NS,), jnp.float32),
            pltpu.VMEM((_C, _D), jnp.float32),
            pltpu.VMEM((_C, _D), jnp.float32),
            pltpu.VMEM((_C,), jnp.int32),
            pltpu.VMEM((_C,), jnp.int32),
            pltpu.VMEM((_C,), jnp.int32),
            pltpu.VMEM((_C,), jnp.int32),
            pltpu.VMEM((_C,), jnp.int32),
            pltpu.VMEM((_C,), jnp.int32),
            pltpu.VMEM((_C,), jnp.float32),
            pltpu.VMEM((_C,), jnp.float32),
            pltpu.VMEM_SHARED((_N, _D), jnp.float32),
            pltpu.VMEM_SHARED((_RNP,), jnp.float32),
            pltpu.SemaphoreType.DMA,
            pltpu.SemaphoreType.DMA,
            pltpu.SemaphoreType.DMA,
            pltpu.SemaphoreType.DMA,
        ],
        compiler_params=pltpu.CompilerParams(needs_layout_passes=False),
    )(tt, degf[0], degf[1], srcf, dstf, etf)

    out = pl.pallas_call(
        _fin_body,
        grid=(_N // _BN,),
        in_specs=[
            pl.BlockSpec((_NC, _BN, _D), lambda i: (0, i, 0)),
            pl.BlockSpec((1, _D), lambda i: (0, 0)),
        ],
        out_specs=pl.BlockSpec((_BN, _D), lambda i: (i, 0)),
        out_shape=jax.ShapeDtypeStruct((_N, _D), jnp.float32),
    )(outp, b.reshape(1, _D))
    return out
